# baseline (speedup 1.0000x reference)
import functools

import jax
import jax.numpy as jnp
from jax.experimental import pallas as pl
from jax.experimental.pallas import tpu as pltpu

_HIDDEN = 768
_INTER = 1536
_N_SHARED = 2
_N_ROUTED = 8
_TOP_K = 2
_T_BLK = 256
_F_BLK = 768


def _moe_body(x_ref, gw_ref, wg_ref, wu_ref, wd_ref, out_ref, w_scr):
    e = pl.program_id(1)
    f = pl.program_id(2)

    x = x_ref[...]

    @pl.when(jnp.logical_and(e == 0, f == 0))
    def _router():
        logits = jax.lax.dot_general(
            x, gw_ref[...], (((1,), (1,)), ((), ())),
            preferred_element_type=jnp.float32)
        p = jax.nn.softmax(logits, axis=-1)
        i1 = jnp.argmax(p, axis=-1)
        lane = jax.lax.broadcasted_iota(jnp.int32, p.shape, 1)
        m1h = (lane == i1[:, None])
        m1 = jnp.max(p, axis=-1, keepdims=True)
        p2 = jnp.where(m1h, -jnp.inf, p)
        i2 = jnp.argmax(p2, axis=-1)
        m2h = (lane == i2[:, None])
        m2 = jnp.max(p2, axis=-1, keepdims=True)
        denom = m1 + m2 + 1e-8
        w_scr[...] = jnp.where(m1h | m2h, p / denom, 0.0)

    @pl.when(jnp.logical_and(e == 0, f == 0))
    def _init():
        out_ref[...] = jnp.zeros_like(out_ref)

    onehot = (jax.lax.broadcasted_iota(jnp.int32, (_T_BLK, _N_ROUTED), 1)
              == e)
    w_col = jnp.sum(jnp.where(onehot, w_scr[...], 0.0), axis=1,
                    keepdims=True)
    w_col = jnp.where(e < _N_ROUTED, w_col, 1.0 / _N_SHARED)

    g = jax.lax.dot_general(x, wg_ref[0], (((1,), (1,)), ((), ())),
                            preferred_element_type=jnp.float32)
    u = jax.lax.dot_general(x, wu_ref[0], (((1,), (1,)), ((), ())),
                            preferred_element_type=jnp.float32)
    gu = jax.nn.silu(g) * u * w_col
    d = jax.lax.dot_general(gu, wd_ref[0], (((1,), (1,)), ((), ())),
                            preferred_element_type=jnp.float32)
    out_ref[...] += d


@functools.partial(jax.jit, static_argnames=())
def kernel(hidden_states, gate_W, sWg, sWu, sWd, rWg, rWu, rWd):
    b, s, h = hidden_states.shape
    xf = hidden_states.reshape(s, h)
    Wg = jnp.concatenate([rWg, sWg], axis=0)
    Wu = jnp.concatenate([rWu, sWu], axis=0)
    Wd = jnp.concatenate([rWd, sWd], axis=0)

    n_e = _N_ROUTED + _N_SHARED
    grid = (s // _T_BLK, n_e, _INTER // _F_BLK)

    out = pl.pallas_call(
        _moe_body,
        grid=grid,
        in_specs=[
            pl.BlockSpec((_T_BLK, h), lambda t, e, f: (t, 0)),
            pl.BlockSpec((_N_ROUTED, h), lambda t, e, f: (0, 0)),
            pl.BlockSpec((1, _F_BLK, h), lambda t, e, f: (e, f, 0)),
            pl.BlockSpec((1, _F_BLK, h), lambda t, e, f: (e, f, 0)),
            pl.BlockSpec((1, h, _F_BLK), lambda t, e, f: (e, 0, f)),
        ],
        out_specs=pl.BlockSpec((_T_BLK, h), lambda t, e, f: (t, 0)),
        out_shape=jax.ShapeDtypeStruct((s, h), jnp.float32),
        scratch_shapes=[pltpu.VMEM((_T_BLK, _N_ROUTED), jnp.float32)],
        compiler_params=pltpu.CompilerParams(
            dimension_semantics=("parallel", "arbitrary", "arbitrary"),
        ),
    )(xf, gate_W, Wg, Wu, Wd)
    return out.reshape(b, s, h)


# speedup vs baseline: 1.1902x; 1.1902x over previous
import functools

import jax
import jax.numpy as jnp
from jax.experimental import pallas as pl
from jax.experimental.pallas import tpu as pltpu

_HIDDEN = 768
_INTER = 1536
_N_SHARED = 2
_N_ROUTED = 8
_TOP_K = 2
_T_BLK = 256
_F_BLK = 768


def _moe_body(x_ref, gw_ref, wg_ref, wu_ref, wd_ref, out_ref, w_scr):
    e = pl.program_id(1)
    f = pl.program_id(2)

    x = x_ref[...]

    @pl.when(jnp.logical_and(e == 0, f == 0))
    def _router():
        logits = jax.lax.dot_general(
            x, gw_ref[...], (((1,), (1,)), ((), ())),
            preferred_element_type=jnp.float32)
        p = jax.nn.softmax(logits, axis=-1)
        i1 = jnp.argmax(p, axis=-1)
        lane = jax.lax.broadcasted_iota(jnp.int32, p.shape, 1)
        m1h = (lane == i1[:, None])
        m1 = jnp.max(p, axis=-1, keepdims=True)
        p2 = jnp.where(m1h, -jnp.inf, p)
        i2 = jnp.argmax(p2, axis=-1)
        m2h = (lane == i2[:, None])
        m2 = jnp.max(p2, axis=-1, keepdims=True)
        denom = m1 + m2 + 1e-8
        w_scr[...] = jnp.where(m1h | m2h, p / denom, 0.0)

    @pl.when(jnp.logical_and(e == 0, f == 0))
    def _init():
        out_ref[...] = jnp.zeros_like(out_ref)

    onehot = (jax.lax.broadcasted_iota(jnp.int32, (_T_BLK, _N_ROUTED), 1)
              == e)
    w_col = jnp.sum(jnp.where(onehot, w_scr[...], 0.0), axis=1,
                    keepdims=True)
    w_col = jnp.where(e < _N_ROUTED, w_col, 1.0 / _N_SHARED)

    xb = x.astype(jnp.bfloat16)
    g = jax.lax.dot_general(xb, wg_ref[0], (((1,), (1,)), ((), ())),
                            preferred_element_type=jnp.float32)
    u = jax.lax.dot_general(xb, wu_ref[0], (((1,), (1,)), ((), ())),
                            preferred_element_type=jnp.float32)
    gu = (jax.nn.silu(g) * u * w_col).astype(jnp.bfloat16)
    d = jax.lax.dot_general(gu, wd_ref[0], (((1,), (1,)), ((), ())),
                            preferred_element_type=jnp.float32)
    out_ref[...] += d


@functools.partial(jax.jit, static_argnames=())
def kernel(hidden_states, gate_W, sWg, sWu, sWd, rWg, rWu, rWd):
    b, s, h = hidden_states.shape
    xf = hidden_states.reshape(s, h)
    Wg = jnp.concatenate([rWg, sWg], axis=0).astype(jnp.bfloat16)
    Wu = jnp.concatenate([rWu, sWu], axis=0).astype(jnp.bfloat16)
    Wd = jnp.concatenate([rWd, sWd], axis=0).astype(jnp.bfloat16)

    n_e = _N_ROUTED + _N_SHARED
    grid = (s // _T_BLK, n_e, _INTER // _F_BLK)

    out = pl.pallas_call(
        _moe_body,
        grid=grid,
        in_specs=[
            pl.BlockSpec((_T_BLK, h), lambda t, e, f: (t, 0)),
            pl.BlockSpec((_N_ROUTED, h), lambda t, e, f: (0, 0)),
            pl.BlockSpec((1, _F_BLK, h), lambda t, e, f: (e, f, 0)),
            pl.BlockSpec((1, _F_BLK, h), lambda t, e, f: (e, f, 0)),
            pl.BlockSpec((1, h, _F_BLK), lambda t, e, f: (e, 0, f)),
        ],
        out_specs=pl.BlockSpec((_T_BLK, h), lambda t, e, f: (t, 0)),
        out_shape=jax.ShapeDtypeStruct((s, h), jnp.float32),
        scratch_shapes=[pltpu.VMEM((_T_BLK, _N_ROUTED), jnp.float32)],
        compiler_params=pltpu.CompilerParams(
            dimension_semantics=("parallel", "arbitrary", "arbitrary"),
        ),
    )(xf, gate_W, Wg, Wu, Wd)
    return out.reshape(b, s, h)


# speedup vs baseline: 1.5650x; 1.3149x over previous
import functools

import jax
import jax.numpy as jnp
from jax.experimental import pallas as pl
from jax.experimental.pallas import tpu as pltpu

_HIDDEN = 768
_INTER = 1536
_N_SHARED = 2
_N_ROUTED = 8
_TOP_K = 2
_F_BLK = 768


def _moe_body(x_ref, gw_ref, wg_ref, wu_ref, wd_ref, out_ref, w_scr):
    e = pl.program_id(0)
    f = pl.program_id(1)

    x = x_ref[...]
    T = x.shape[0]

    @pl.when(jnp.logical_and(e == 0, f == 0))
    def _router():
        logits = jax.lax.dot_general(
            x, gw_ref[...], (((1,), (1,)), ((), ())),
            preferred_element_type=jnp.float32)
        p = jax.nn.softmax(logits, axis=-1)
        i1 = jnp.argmax(p, axis=-1)
        lane = jax.lax.broadcasted_iota(jnp.int32, p.shape, 1)
        m1h = (lane == i1[:, None])
        m1 = jnp.max(p, axis=-1, keepdims=True)
        p2 = jnp.where(m1h, -jnp.inf, p)
        i2 = jnp.argmax(p2, axis=-1)
        m2h = (lane == i2[:, None])
        m2 = jnp.max(p2, axis=-1, keepdims=True)
        denom = m1 + m2 + 1e-8
        w_scr[...] = jnp.where(m1h | m2h, p / denom, 0.0)
        out_ref[...] = jnp.zeros_like(out_ref)

    onehot = (jax.lax.broadcasted_iota(jnp.int32, (2048, _N_ROUTED), 1)
              == e)
    w_col = jnp.sum(jnp.where(onehot, w_scr[...], 0.0), axis=1,
                    keepdims=True)
    w_col = jnp.where(e < _N_ROUTED, w_col, 1.0 / _N_SHARED)

    xb = x.astype(jnp.bfloat16)
    g = jax.lax.dot_general(xb, wg_ref[0], (((1,), (1,)), ((), ())),
                            preferred_element_type=jnp.float32)
    u = jax.lax.dot_general(xb, wu_ref[0], (((1,), (1,)), ((), ())),
                            preferred_element_type=jnp.float32)
    gu = (jax.nn.silu(g) * u * w_col).astype(jnp.bfloat16)
    d = jax.lax.dot_general(gu, wd_ref[0], (((1,), (1,)), ((), ())),
                            preferred_element_type=jnp.float32)
    out_ref[...] += d


@functools.partial(jax.jit, static_argnames=())
def kernel(hidden_states, gate_W, sWg, sWu, sWd, rWg, rWu, rWd):
    b, s, h = hidden_states.shape
    xf = hidden_states.reshape(s, h)
    Wg = jnp.concatenate([rWg, sWg], axis=0).astype(jnp.bfloat16)
    Wu = jnp.concatenate([rWu, sWu], axis=0).astype(jnp.bfloat16)
    Wd = jnp.concatenate([rWd, sWd], axis=0).astype(jnp.bfloat16)

    n_e = _N_ROUTED + _N_SHARED
    grid = (n_e, _INTER // _F_BLK)

    out = pl.pallas_call(
        _moe_body,
        grid=grid,
        in_specs=[
            pl.BlockSpec((s, h), lambda e, f: (0, 0)),
            pl.BlockSpec((_N_ROUTED, h), lambda e, f: (0, 0)),
            pl.BlockSpec((1, _F_BLK, h), lambda e, f: (e, f, 0)),
            pl.BlockSpec((1, _F_BLK, h), lambda e, f: (e, f, 0)),
            pl.BlockSpec((1, h, _F_BLK), lambda e, f: (e, 0, f)),
        ],
        out_specs=pl.BlockSpec((s, h), lambda e, f: (0, 0)),
        out_shape=jax.ShapeDtypeStruct((s, h), jnp.float32),
        scratch_shapes=[pltpu.VMEM((s, _N_ROUTED), jnp.float32)],
        compiler_params=pltpu.CompilerParams(
            dimension_semantics=("arbitrary", "arbitrary"),
        ),
    )(xf, gate_W, Wg, Wu, Wd)
    return out.reshape(b, s, h)
